# baseline (speedup 1.0000x reference)
import jax
import jax.numpy as jnp
from jax.experimental import pallas as pl
from jax.experimental.pallas import tpu as pltpu

N = 5000
D = 4096
C = 81
CW = C + 4
TILE = 1000


def _head_kernel(fa_ref, fb_ref, p_ref, wa_ref, wb_ref, b_ref,
                 cls_ref, box_ref):
    acc = jnp.dot(fa_ref[...], wa_ref[...],
                  preferred_element_type=jnp.float32)
    acc = acc + jnp.dot(fb_ref[...], wb_ref[...],
                        preferred_element_type=jnp.float32)
    acc = acc + b_ref[...]
    cls_ref[...] = acc[:, :C]

    deltas = acc[:, C:CW]
    p = p_ref[...]
    px, py = p[:, 0:1], p[:, 1:2]
    pw, ph = p[:, 2:3], p[:, 3:4]
    x = deltas[:, 0:1] * pw + px
    y = deltas[:, 1:2] * ph + py
    ew = jnp.exp(deltas[:, 2:3])
    w = ew * pw
    h = ew * ph
    box_ref[...] = jnp.concatenate([x, y, w, h], axis=1)


def kernel(feats, proposals_xywh, W_cls, b_cls, W_reg, b_reg):
    w_t = jnp.concatenate([W_cls, W_reg], axis=0).T
    b = jnp.concatenate([b_cls, b_reg]).reshape(1, CW)
    grid = (N // TILE,)
    H = D // 2
    cls_out, box_out = pl.pallas_call(
        _head_kernel,
        grid=grid,
        in_specs=[
            pl.BlockSpec((TILE, H), lambda i: (i, 0)),
            pl.BlockSpec((TILE, H), lambda i: (i, 1)),
            pl.BlockSpec((TILE, 4), lambda i: (i, 0)),
            pl.BlockSpec((H, CW), lambda i: (0, 0)),
            pl.BlockSpec((H, CW), lambda i: (1, 0)),
            pl.BlockSpec((1, CW), lambda i: (0, 0)),
        ],
        out_specs=[
            pl.BlockSpec((TILE, C), lambda i: (i, 0)),
            pl.BlockSpec((TILE, 4), lambda i: (i, 0)),
        ],
        out_shape=[
            jax.ShapeDtypeStruct((N, C), jnp.float32),
            jax.ShapeDtypeStruct((N, 4), jnp.float32),
        ],
        compiler_params=pltpu.CompilerParams(
            dimension_semantics=("parallel",)),
    )(feats, feats, proposals_xywh, w_t, w_t, b)
    return (cls_out, box_out)
